# Initial kernel scaffold; baseline (speedup 1.0000x reference)
import jax
import jax.numpy as jnp
from jax.experimental import pallas as pl


def kernel(x_src, x_dst, e, W, b):
    raise NotImplementedError("write your pallas kernel here")


# speedup vs baseline: 1.8808x; 1.8808x over previous
import functools

import jax
import jax.numpy as jnp
from jax import lax
from jax.experimental import pallas as pl
from jax.experimental.pallas import tpu as pltpu
from jax.experimental.pallas import tpu_sc as plsc

N = 10000
E = 160000
D = 256

NC = 2
NS = 16
L = 16
NW = NC * NS
NPT = (N + NW - 1) // NW
NPAD = NPT * NW

C = 3200
NCHUNKS = E // C
G = 32
SENTINEL = 3.0e38

_mesh = plsc.VectorSubcoreMesh(
    core_axis_name="c", subcore_axis_name="s", num_cores=NC, num_subcores=NS
)


@functools.partial(
    pl.kernel,
    out_type=jax.ShapeDtypeStruct((NPAD * D,), jnp.float32),
    mesh=_mesh,
    scratch_types=[
        pltpu.VMEM((NPT * D,), jnp.float32),
        pltpu.VMEM((C,), jnp.int32),
        pltpu.VMEM((C,), jnp.int32),
        pltpu.VMEM((C + L,), jnp.int32),
        pltpu.VMEM((C + L,), jnp.int32),
        pltpu.VMEM((G, D), jnp.float32),
        pltpu.SemaphoreType.DMA,
    ],
    compiler_params=pltpu.CompilerParams(needs_layout_passes=False),
)
def _segmin_kernel(e_hbm, xsrc_hbm, out_hbm, acc, srcb, dstb, csrc, cdst, rows, sem):
    wid = lax.axis_index("s") * NC + lax.axis_index("c")
    lo = wid * NPT

    big = jnp.full((L,), SENTINEL, dtype=jnp.float32)

    def init_acc(i, carry):
        acc[pl.ds(i * L, L)] = big
        return carry

    lax.fori_loop(0, NPT * D // L, init_acc, 0)

    zero = jnp.zeros((L,), dtype=jnp.int32)

    def init_csrc(i, carry):
        csrc[pl.ds(i * L, L)] = zero
        return carry

    lax.fori_loop(0, (C + L) // L, init_csrc, 0)

    def chunk_body(ci, carry):
        off = ci * C
        pltpu.sync_copy(e_hbm.at[pl.ds(off, C)], srcb)
        pltpu.sync_copy(e_hbm.at[pl.ds(E + off, C)], dstb)

        def scan_body(i, cnt):
            d = dstb[pl.ds(i * L, L)]
            s = srcb[pl.ds(i * L, L)]
            dl = d - lo
            mask = (dl >= 0) & (dl < NPT)
            csum = plsc.cumsum(jnp.where(mask, 1, 0))
            pos = cnt + csum - 1
            plsc.store_scatter(csrc, [pos], s, mask=mask)
            plsc.store_scatter(cdst, [pos], dl, mask=mask)
            return cnt + csum[L - 1]

        cnt = lax.fori_loop(0, C // L, scan_body, 0)

        nb = (cnt + G - 1) // G

        def gather_body(g, carry2):
            pltpu.async_copy(xsrc_hbm.at[csrc.at[pl.ds(g * G, G)]], rows, sem).wait()
            lim = cnt - g * G

            def edge_body(j, carry3):
                @pl.when(j < lim)
                def _apply():
                    ldst = cdst[pl.ds(g * G + j, L)][0]
                    base = ldst * D
                    for cc in range(D // L):
                        sl = pl.ds(base + cc * L, L)
                        acc[sl] = jnp.minimum(acc[sl], rows[j, pl.ds(cc * L, L)])

                return carry3

            lax.fori_loop(0, G, edge_body, 0)
            return carry2

        lax.fori_loop(0, nb, gather_body, 0)
        return carry

    lax.fori_loop(0, NCHUNKS, chunk_body, 0)

    pltpu.sync_copy(acc, out_hbm.at[pl.ds(lo * D, NPT * D)])


BN = 400


def _mlp_body(xd_ref, m_ref, w_ref, b_ref, out_ref):
    xd = xd_ref[...]
    m = m_ref[...]
    maxes = jnp.where(m > 1e38, 0.0, xd - m)
    w1 = w_ref[:, :D]
    w2 = w_ref[:, D:]
    h = lax.dot_general(xd, w1, (((1,), (1,)), ((), ())),
                        preferred_element_type=jnp.float32)
    h = h + lax.dot_general(maxes, w2, (((1,), (1,)), ((), ())),
                            preferred_element_type=jnp.float32)
    h = h + b_ref[...]
    h = jnp.where(h >= 0, h, 0.01 * h)
    out_ref[...] = xd + h


def kernel(x_src, x_dst, e, W, b):
    m_flat = _segmin_kernel(e.reshape(-1), x_src)
    m = m_flat.reshape(NPAD, D)[:N]

    out = pl.pallas_call(
        _mlp_body,
        grid=(N // BN,),
        in_specs=[
            pl.BlockSpec((BN, D), lambda i: (i, 0)),
            pl.BlockSpec((BN, D), lambda i: (i, 0)),
            pl.BlockSpec((D, 2 * D), lambda i: (0, 0)),
            pl.BlockSpec((1, D), lambda i: (0, 0)),
        ],
        out_specs=pl.BlockSpec((BN, D), lambda i: (i, 0)),
        out_shape=jax.ShapeDtypeStruct((N, D), jnp.float32),
    )(x_dst, m, W, b.reshape(1, D))
    return out
